# baseline (speedup 1.0000x reference)
import functools

import jax
import jax.numpy as jnp
from jax import lax
from jax.experimental import pallas as pl
from jax.experimental.pallas import tpu as pltpu
from jax.experimental.pallas import tpu_sc as plsc

_B = 64
_L = 128
_DM = 768
_DF = 1536
_DL = 2048
_E = 8

_HW = _DM // 4
_HO = _DF // 2
_SPB = 2
_ND = 3
_SPT = 2


def _logits_kernel(dkp_ref, cn_ref, w1_ref, b1_ref, w2_ref, b2_ref, w3_ref,
                   b3_ref, out_ref):
    h1 = jnp.maximum(
        jnp.dot(dkp_ref[...], w1_ref[...], preferred_element_type=jnp.float32)
        + b1_ref[...], 0.0)
    h2 = jnp.maximum(cn_ref[...] * w2_ref[...] + b2_ref[...], 0.0)
    h = h1 + h2
    logits = (jnp.dot(h, w3_ref[...], preferred_element_type=jnp.float32)
              + b3_ref[...])
    pad = jnp.full((_B, 16 - _E), -jnp.inf, jnp.float32)
    out_ref[...] = jnp.concatenate([logits, pad], axis=1)


def _allreduce(op, v, iota):
    for k in (1, 2, 4, 8):
        perm = jnp.bitwise_xor(iota, k)
        v = op(v, v.at[perm].get(mode="promise_in_bounds"))
    return v


def _route_sc_body(lg_hbm, ee_hbm, gates_hbm, lg_v, ee_v, g_v):
    wid = lax.axis_index("s") * 2 + lax.axis_index("c")

    @pl.when(wid < _B // _SPT)
    def _():
        base = wid * _SPT
        pltpu.sync_copy(lg_hbm.at[pl.ds(base, _SPT)], lg_v)
        iota = lax.iota(jnp.int32, 16)
        for i in range(_SPT):
            v = lg_v[i]
            m1 = _allreduce(jnp.maximum, v, iota)
            i1 = _allreduce(jnp.minimum,
                            jnp.where(v == m1, iota, 16), iota)
            masked = jnp.where(iota == i1, -jnp.inf, v)
            m2 = _allreduce(jnp.maximum, masked, iota)
            i2 = _allreduce(jnp.minimum,
                            jnp.where(masked == m2, iota, 16), iota)
            p = jnp.exp(v - m1)
            p = p / _allreduce(jnp.add, p, iota)
            sel = (iota == i1) | (iota == i2)
            pm = jnp.where(sel, p, 0.0)
            g_v[i] = pm / (_allreduce(jnp.add, pm, iota) + 1e-9)
            ee_v[i] = jnp.where(iota == 0, i1,
                                jnp.where(iota == 1, i2, 0))
        pltpu.sync_copy(g_v, gates_hbm.at[pl.ds(base, _SPT)])
        pltpu.sync_copy(ee_v, ee_hbm.at[pl.ds(base, _SPT)])


def _route_sc(logits16):
    mesh = plsc.VectorSubcoreMesh(core_axis_name="c", subcore_axis_name="s")
    fn = functools.partial(
        pl.kernel,
        mesh=mesh,
        out_type=(
            jax.ShapeDtypeStruct((_B, 16), jnp.int32),
            jax.ShapeDtypeStruct((_B, 16), jnp.float32),
        ),
        scratch_types=[
            pltpu.VMEM((_SPT, 16), jnp.float32),
            pltpu.VMEM((_SPT, 16), jnp.int32),
            pltpu.VMEM((_SPT, 16), jnp.float32),
        ],
    )(_route_sc_body)
    return fn(logits16)


def _moe_kernel(x_ref, gates_ref, ee_ref, ewi_ref, ewo_ref, gwi_ref, gwo_ref,
                bi_ref, bo_ref, lg_ref, lb_ref, out_ref,
                wi_bf, wo_bf, stg_i, stg_o, ee_smem, sem_i, sem_o, sem_ee):
    s = pl.program_id(0)

    @pl.when(s == 0)
    def _prologue():
        ee_dma = pltpu.make_async_copy(ee_ref, ee_smem, sem_ee.at[0])
        ee_dma.start()

        wi_srcs = ([(ewi_ref, e, h, e) for e in range(_E) for h in range(4)]
                   + [(gwi_ref, 0, h, _E) for h in range(4)])
        wo_srcs = ([(ewo_ref, e, h, e) for e in range(_E) for h in range(2)]
                   + [(gwo_ref, 0, h, _E) for h in range(2)])

        def wi_cp(k):
            src, se, h, _ = wi_srcs[k]
            return pltpu.make_async_copy(
                src.at[se, pl.ds(h * _HW, _HW), :], stg_i.at[k % _ND],
                sem_i.at[k % _ND])

        def wo_cp(k):
            src, se, h, _ = wo_srcs[k]
            return pltpu.make_async_copy(
                src.at[se, pl.ds(h * _HO, _HO), :], stg_o.at[k % _ND],
                sem_o.at[k % _ND])

        ni = len(wi_srcs)
        no = len(wo_srcs)
        for d in range(_ND - 1):
            if d < ni:
                wi_cp(d).start()
            if d < no:
                wo_cp(d).start()

        for k in range(ni):
            if k + _ND - 1 < ni:
                wi_cp(k + _ND - 1).start()
            if k % 2 == 0 and k // 2 + _ND - 1 < no:
                wo_cp(k // 2 + _ND - 1).start()
            wi_cp(k).wait()
            _, _, h_, de = wi_srcs[k]
            wi_bf[de, pl.ds(h_ * _HW, _HW), :] = stg_i[k % _ND].astype(
                jnp.bfloat16)
            if k % 2 == 1:
                ko = k // 2
                wo_cp(ko).wait()
                _, _, h2_, de2 = wo_srcs[ko]
                wo_bf[de2, pl.ds(h2_ * _HO, _HO), :] = stg_o[ko % _ND].astype(
                    jnp.bfloat16)
        ee_dma.wait()

    def mlp_of(xv, xbv, e):
        h = jnp.maximum(
            jnp.dot(xbv, wi_bf[e], preferred_element_type=jnp.float32)
            + bi_ref[e], 0.0)
        o = (jnp.dot(h.astype(jnp.bfloat16), wo_bf[e],
                     preferred_element_type=jnp.float32)
             + bo_ref[e] + xv)
        mu = jnp.mean(o, axis=1, keepdims=True)
        var = jnp.mean((o - mu) ** 2, axis=1, keepdims=True)
        return (o - mu) / jnp.sqrt(var + 1e-5) * lg_ref[e] + lb_ref[e]

    xf = x_ref[...].reshape(_SPB * _L, _DM)
    xbf = xf.astype(jnp.bfloat16)
    gen = mlp_of(xf, xbf, _E)

    lane16 = jax.lax.broadcasted_iota(jnp.int32, (1, 16), 1)
    for i in range(_SPB):
        xi = xf[i * _L:(i + 1) * _L]
        xbi = xbf[i * _L:(i + 1) * _L]
        e0 = ee_smem[s * _SPB + i, 0]
        e1 = ee_smem[s * _SPB + i, 1]
        grow = gates_ref[pl.ds(s * _SPB + i, 1), :]
        w0 = jnp.sum(jnp.where(lane16 == e0, grow, 0.0))
        w1 = jnp.sum(jnp.where(lane16 == e1, grow, 0.0))
        tot = mlp_of(xi, xbi, e0) * w0 + mlp_of(xi, xbi, e1) * w1
        tot = tot.astype(jnp.bfloat16).astype(jnp.float32)
        out_ref[i] = gen[i * _L:(i + 1) * _L] + tot


def kernel(cycle_curve_data, cycle_numbers, DKP_embeddings, gate_W1, gate_b1,
           gate_W2, gate_b2, gate_W3, gate_b3, exp_Wi, exp_bi, exp_Wo, exp_bo,
           exp_g, exp_b, gen_Wi, gen_bi, gen_Wo, gen_bo, gen_g, gen_b):
    logits16 = pl.pallas_call(
        _logits_kernel,
        out_shape=jax.ShapeDtypeStruct((_B, 16), jnp.float32),
    )(DKP_embeddings, cycle_numbers, gate_W1, gate_b1.reshape(1, _DF),
      gate_W2, gate_b2.reshape(1, _DF), gate_W3, gate_b3.reshape(1, _E))

    ee16, gates16 = _route_sc(logits16)

    bi_all = jnp.concatenate([exp_bi, gen_bi[None]], axis=0)[:, None, :]
    bo_all = jnp.concatenate([exp_bo, gen_bo[None]], axis=0)[:, None, :]
    lg_all = jnp.concatenate([exp_g, gen_g[None]], axis=0)[:, None, :]
    lb_all = jnp.concatenate([exp_b, gen_b[None]], axis=0)[:, None, :]

    final = pl.pallas_call(
        _moe_kernel,
        grid=(_B // _SPB,),
        in_specs=[
            pl.BlockSpec((_SPB, _L, _DM), lambda s: (s, 0, 0)),
            pl.BlockSpec((_B, 16), lambda s: (0, 0)),
            pl.BlockSpec(memory_space=pl.ANY),
            pl.BlockSpec(memory_space=pl.ANY),
            pl.BlockSpec(memory_space=pl.ANY),
            pl.BlockSpec(memory_space=pl.ANY),
            pl.BlockSpec(memory_space=pl.ANY),
            pl.BlockSpec((_E + 1, 1, _DF), lambda s: (0, 0, 0)),
            pl.BlockSpec((_E + 1, 1, _DM), lambda s: (0, 0, 0)),
            pl.BlockSpec((_E + 1, 1, _DM), lambda s: (0, 0, 0)),
            pl.BlockSpec((_E + 1, 1, _DM), lambda s: (0, 0, 0)),
        ],
        out_specs=pl.BlockSpec((_SPB, _L, _DM), lambda s: (s, 0, 0)),
        scratch_shapes=[
            pltpu.VMEM((_E + 1, _DM, _DF), jnp.bfloat16),
            pltpu.VMEM((_E + 1, _DF, _DM), jnp.bfloat16),
            pltpu.VMEM((_ND, _HW, _DF), jnp.float32),
            pltpu.VMEM((_ND, _HO, _DM), jnp.float32),
            pltpu.SMEM((_B, 16), jnp.int32),
            pltpu.SemaphoreType.DMA((_ND,)),
            pltpu.SemaphoreType.DMA((_ND,)),
            pltpu.SemaphoreType.DMA((1,)),
        ],
        out_shape=jax.ShapeDtypeStruct((_B, _L, _DM), jnp.float32),
    )(cycle_curve_data, gates16, ee16, exp_Wi, exp_Wo,
      gen_Wi.reshape(1, _DM, _DF), gen_Wo.reshape(1, _DF, _DM),
      bi_all, bo_all, lg_all, lb_all)

    return (final, jnp.float32(0.0))


# speedup vs baseline: 1.0017x; 1.0017x over previous
import functools

import jax
import jax.numpy as jnp
from jax import lax
from jax.experimental import pallas as pl
from jax.experimental.pallas import tpu as pltpu
from jax.experimental.pallas import tpu_sc as plsc

_B = 64
_L = 128
_DM = 768
_DF = 1536
_DL = 2048
_E = 8

_HW = _DM // 4
_HO = _DF // 4
_SPB = 2
_ND = 4
_SPT = 2


def _logits_kernel(dkp_ref, cn_ref, w1_ref, b1_ref, w2_ref, b2_ref, w3_ref,
                   b3_ref, out_ref):
    h1 = jnp.maximum(
        jnp.dot(dkp_ref[...], w1_ref[...], preferred_element_type=jnp.float32)
        + b1_ref[...], 0.0)
    h2 = jnp.maximum(cn_ref[...] * w2_ref[...] + b2_ref[...], 0.0)
    h = h1 + h2
    logits = (jnp.dot(h, w3_ref[...], preferred_element_type=jnp.float32)
              + b3_ref[...])
    pad = jnp.full((_B, 16 - _E), -jnp.inf, jnp.float32)
    out_ref[...] = jnp.concatenate([logits, pad], axis=1)


def _allreduce(op, v, iota):
    for k in (1, 2, 4, 8):
        perm = jnp.bitwise_xor(iota, k)
        v = op(v, v.at[perm].get(mode="promise_in_bounds"))
    return v


def _route_sc_body(lg_hbm, ee_hbm, gates_hbm, lg_v, ee_v, g_v):
    wid = lax.axis_index("s") * 2 + lax.axis_index("c")

    @pl.when(wid < _B // _SPT)
    def _():
        base = wid * _SPT
        pltpu.sync_copy(lg_hbm.at[pl.ds(base, _SPT)], lg_v)
        iota = lax.iota(jnp.int32, 16)
        for i in range(_SPT):
            v = lg_v[i]
            m1 = _allreduce(jnp.maximum, v, iota)
            i1 = _allreduce(jnp.minimum,
                            jnp.where(v == m1, iota, 16), iota)
            masked = jnp.where(iota == i1, -jnp.inf, v)
            m2 = _allreduce(jnp.maximum, masked, iota)
            i2 = _allreduce(jnp.minimum,
                            jnp.where(masked == m2, iota, 16), iota)
            p = jnp.exp(v - m1)
            p = p / _allreduce(jnp.add, p, iota)
            sel = (iota == i1) | (iota == i2)
            pm = jnp.where(sel, p, 0.0)
            g_v[i] = pm / (_allreduce(jnp.add, pm, iota) + 1e-9)
            ee_v[i] = jnp.where(iota == 0, i1,
                                jnp.where(iota == 1, i2, 0))
        pltpu.sync_copy(g_v, gates_hbm.at[pl.ds(base, _SPT)])
        pltpu.sync_copy(ee_v, ee_hbm.at[pl.ds(base, _SPT)])


def _route_sc(logits16):
    mesh = plsc.VectorSubcoreMesh(core_axis_name="c", subcore_axis_name="s")
    fn = functools.partial(
        pl.kernel,
        mesh=mesh,
        out_type=(
            jax.ShapeDtypeStruct((_B, 16), jnp.int32),
            jax.ShapeDtypeStruct((_B, 16), jnp.float32),
        ),
        scratch_types=[
            pltpu.VMEM((_SPT, 16), jnp.float32),
            pltpu.VMEM((_SPT, 16), jnp.int32),
            pltpu.VMEM((_SPT, 16), jnp.float32),
        ],
    )(_route_sc_body)
    return fn(logits16)


def _moe_kernel(x_ref, gates_ref, ee_ref, ewi_ref, ewo_ref, gwi_ref, gwo_ref,
                bi_ref, bo_ref, lg_ref, lb_ref, out_ref,
                wi_bf, wo_bf, stg_i, stg_o, ee_smem, sem_i, sem_o, sem_ee):
    s = pl.program_id(0)

    @pl.when(s == 0)
    def _prologue():
        ee_dma = pltpu.make_async_copy(ee_ref, ee_smem, sem_ee.at[0])
        ee_dma.start()

        wi_srcs = ([(ewi_ref, e, h, e) for e in range(_E) for h in range(4)]
                   + [(gwi_ref, 0, h, _E) for h in range(4)])
        wo_srcs = ([(ewo_ref, e, h, e) for e in range(_E) for h in range(4)]
                   + [(gwo_ref, 0, h, _E) for h in range(4)])

        def wi_cp(k):
            src, se, h, _ = wi_srcs[k]
            return pltpu.make_async_copy(
                src.at[se, pl.ds(h * _HW, _HW), :], stg_i.at[k % _ND],
                sem_i.at[k % _ND])

        def wo_cp(k):
            src, se, h, _ = wo_srcs[k]
            return pltpu.make_async_copy(
                src.at[se, pl.ds(h * _HO, _HO), :], stg_o.at[k % _ND],
                sem_o.at[k % _ND])

        ni = len(wi_srcs)
        no = len(wo_srcs)
        for d in range(_ND - 1):
            if d < ni:
                wi_cp(d).start()
            if d < no:
                wo_cp(d).start()

        for k in range(ni):
            if k + _ND - 1 < ni:
                wi_cp(k + _ND - 1).start()
            if k + _ND - 1 < no:
                wo_cp(k + _ND - 1).start()
            wi_cp(k).wait()
            _, _, h_, de = wi_srcs[k]
            wi_bf[de, pl.ds(h_ * _HW, _HW), :] = stg_i[k % _ND].astype(
                jnp.bfloat16)
            wo_cp(k).wait()
            _, _, h2_, de2 = wo_srcs[k]
            wo_bf[de2, pl.ds(h2_ * _HO, _HO), :] = stg_o[k % _ND].astype(
                jnp.bfloat16)
        ee_dma.wait()

    def mlp_of(xv, xbv, e):
        h = jnp.maximum(
            jnp.dot(xbv, wi_bf[e], preferred_element_type=jnp.float32)
            + bi_ref[e], 0.0)
        o = (jnp.dot(h.astype(jnp.bfloat16), wo_bf[e],
                     preferred_element_type=jnp.float32)
             + bo_ref[e] + xv)
        mu = jnp.mean(o, axis=1, keepdims=True)
        var = jnp.mean((o - mu) ** 2, axis=1, keepdims=True)
        return (o - mu) / jnp.sqrt(var + 1e-5) * lg_ref[e] + lb_ref[e]

    xf = x_ref[...].reshape(_SPB * _L, _DM)
    xbf = xf.astype(jnp.bfloat16)
    gen = mlp_of(xf, xbf, _E)

    lane16 = jax.lax.broadcasted_iota(jnp.int32, (1, 16), 1)
    for i in range(_SPB):
        xi = xf[i * _L:(i + 1) * _L]
        xbi = xbf[i * _L:(i + 1) * _L]
        e0 = ee_smem[s * _SPB + i, 0]
        e1 = ee_smem[s * _SPB + i, 1]
        grow = gates_ref[pl.ds(s * _SPB + i, 1), :]
        w0 = jnp.sum(jnp.where(lane16 == e0, grow, 0.0))
        w1 = jnp.sum(jnp.where(lane16 == e1, grow, 0.0))
        tot = mlp_of(xi, xbi, e0) * w0 + mlp_of(xi, xbi, e1) * w1
        tot = tot.astype(jnp.bfloat16).astype(jnp.float32)
        out_ref[i] = gen[i * _L:(i + 1) * _L] + tot


def kernel(cycle_curve_data, cycle_numbers, DKP_embeddings, gate_W1, gate_b1,
           gate_W2, gate_b2, gate_W3, gate_b3, exp_Wi, exp_bi, exp_Wo, exp_bo,
           exp_g, exp_b, gen_Wi, gen_bi, gen_Wo, gen_bo, gen_g, gen_b):
    logits16 = pl.pallas_call(
        _logits_kernel,
        out_shape=jax.ShapeDtypeStruct((_B, 16), jnp.float32),
    )(DKP_embeddings, cycle_numbers, gate_W1, gate_b1.reshape(1, _DF),
      gate_W2, gate_b2.reshape(1, _DF), gate_W3, gate_b3.reshape(1, _E))

    ee16, gates16 = _route_sc(logits16)

    bi_all = jnp.concatenate([exp_bi, gen_bi[None]], axis=0)[:, None, :]
    bo_all = jnp.concatenate([exp_bo, gen_bo[None]], axis=0)[:, None, :]
    lg_all = jnp.concatenate([exp_g, gen_g[None]], axis=0)[:, None, :]
    lb_all = jnp.concatenate([exp_b, gen_b[None]], axis=0)[:, None, :]

    final = pl.pallas_call(
        _moe_kernel,
        grid=(_B // _SPB,),
        in_specs=[
            pl.BlockSpec((_SPB, _L, _DM), lambda s: (s, 0, 0)),
            pl.BlockSpec((_B, 16), lambda s: (0, 0)),
            pl.BlockSpec(memory_space=pl.ANY),
            pl.BlockSpec(memory_space=pl.ANY),
            pl.BlockSpec(memory_space=pl.ANY),
            pl.BlockSpec(memory_space=pl.ANY),
            pl.BlockSpec(memory_space=pl.ANY),
            pl.BlockSpec((_E + 1, 1, _DF), lambda s: (0, 0, 0)),
            pl.BlockSpec((_E + 1, 1, _DM), lambda s: (0, 0, 0)),
            pl.BlockSpec((_E + 1, 1, _DM), lambda s: (0, 0, 0)),
            pl.BlockSpec((_E + 1, 1, _DM), lambda s: (0, 0, 0)),
        ],
        out_specs=pl.BlockSpec((_SPB, _L, _DM), lambda s: (s, 0, 0)),
        scratch_shapes=[
            pltpu.VMEM((_E + 1, _DM, _DF), jnp.bfloat16),
            pltpu.VMEM((_E + 1, _DF, _DM), jnp.bfloat16),
            pltpu.VMEM((_ND, _HW, _DF), jnp.float32),
            pltpu.VMEM((_ND, _HO, _DM), jnp.float32),
            pltpu.SMEM((_B, 16), jnp.int32),
            pltpu.SemaphoreType.DMA((_ND,)),
            pltpu.SemaphoreType.DMA((_ND,)),
            pltpu.SemaphoreType.DMA((1,)),
        ],
        out_shape=jax.ShapeDtypeStruct((_B, _L, _DM), jnp.float32),
    )(cycle_curve_data, gates16, ee16, exp_Wi, exp_Wo,
      gen_Wi.reshape(1, _DM, _DF), gen_Wo.reshape(1, _DF, _DM),
      bi_all, bo_all, lg_all, lb_all)

    return (final, jnp.float32(0.0))
